# baseline (speedup 1.0000x reference)
import functools

import jax
import jax.numpy as jnp
from jax import lax
from jax.experimental import pallas as pl
from jax.experimental.pallas import tpu as pltpu
from jax.experimental.pallas import tpu_sc as plsc

_K = 2048
_N = 32768
_B = 128
_L = 16
_NV = _N // _L
_NC = 2
_NS = 16
_NW = _NC * _NS
_RPW = _B // _NW
_HSTRIDE = 2048
_LEVELS = ((20, 2048), (10, 1024), (0, 1024))


def _body(x_hbm, out_hbm, x_v, hist_v, s_v):
    cid = lax.axis_index("c")
    sid = lax.axis_index("s")
    wid = sid * _NC + cid
    lanes = lax.broadcasted_iota(jnp.int32, (_L,), 0)
    lane_base = lanes * _HSTRIDE
    zero16 = jnp.zeros((_L,), jnp.int32)
    ones16 = jnp.ones((_L,), jnp.int32)
    absmask = jnp.int32(0x7FFFFFFF)

    def u_of(j):
        x = x_v[pl.ds(j * _L, _L)]
        return x, lax.bitcast_convert_type(x, jnp.int32) & absmask

    for r in range(_RPW):
        row = wid * _RPW + r
        pltpu.sync_copy(x_hbm.at[row], x_v)

        pref = jnp.int32(0)
        rk = jnp.int32(_K)
        for li, (shift, nbuckets) in enumerate(_LEVELS):
            nch = nbuckets // _L
            bmask = jnp.int32(nbuckets - 1)
            maskshift = shift + (nbuckets - 1).bit_length()

            def zbody(t, _, nch=nch):
                l = t // nch
                j = t - l * nch
                hist_v[pl.ds(l * _HSTRIDE + j * _L, _L)] = zero16
                return 0

            lax.fori_loop(0, 16 * nch, zbody, 0, unroll=4)

            if li == 0:

                def sbody(j, _, shift=shift, bmask=bmask):
                    _, u = u_of(j)
                    bucket = (u >> shift) & bmask
                    plsc.addupdate_scatter(
                        hist_v, [lane_base + bucket], ones16)
                    return 0

            else:

                def sbody(j, _, shift=shift, bmask=bmask,
                          maskshift=maskshift, pref=pref):
                    _, u = u_of(j)
                    bucket = (u >> shift) & bmask
                    m = (u >> maskshift) == (pref >> maskshift)
                    plsc.addupdate_scatter(
                        hist_v, [lane_base + bucket], ones16, mask=m)
                    return 0

            lax.fori_loop(0, _NV, sbody, 0, unroll=4)

            def cbody(jj, carry, nch=nch):
                c = nch - 1 - jj
                csum, cnt = carry
                acc = hist_v[pl.ds(0 * _HSTRIDE + c * _L, _L)]
                for l in range(1, 16):
                    acc = acc + hist_v[pl.ds(l * _HSTRIDE + c * _L, _L)]
                s = plsc.cumsum(lax.rev(acc, dimensions=(0,)))
                s_desc = s + csum
                cnt = cnt + plsc.all_reduce_population_count(s_desc >= rk)
                s_v[pl.ds(c * _L, _L)] = lax.rev(s_desc, dimensions=(0,))
                return csum + jnp.sum(acc), cnt

            _, cntv = lax.fori_loop(
                0, nch, cbody, (jnp.int32(0), zero16), unroll=2)
            b = jnp.max(cntv) - 1
            s_v[pl.ds(nbuckets, _L)] = zero16
            above = jnp.max(plsc.load_gather(s_v, [zero16 + (b + 1)]))
            rk = rk - above
            pref = pref | (b << shift)

        thr = pref

        def fbody(j, _, thr=thr):
            x, u = u_of(j)
            x_v[pl.ds(j * _L, _L)] = jnp.where(u >= thr, x, jnp.float32(0.0))
            return 0

        lax.fori_loop(0, _NV, fbody, 0, unroll=4)
        pltpu.sync_copy(x_v, out_hbm.at[row])


def kernel(X):
    mesh = plsc.VectorSubcoreMesh(
        core_axis_name="c", subcore_axis_name="s",
        num_cores=_NC, num_subcores=_NS)
    return pl.kernel(
        _body,
        out_type=jax.ShapeDtypeStruct((_B, _N), jnp.float32),
        mesh=mesh,
        scratch_types=[
            pltpu.VMEM((_N,), jnp.float32),
            pltpu.VMEM((16 * _HSTRIDE,), jnp.int32),
            pltpu.VMEM((2048 + _L,), jnp.int32),
        ],
        compiler_params=pltpu.CompilerParams(needs_layout_passes=False),
    )(X)


# speedup vs baseline: 3.3255x; 3.3255x over previous
import jax
import jax.numpy as jnp
from jax import lax
from jax.experimental import pallas as pl
from jax.experimental.pallas import tpu as pltpu
from jax.experimental.pallas import tpu_sc as plsc

_K = 2048
_N = 32768
_B = 128
_L = 16
_NV = _N // _L
_NC = 2
_NS = 16
_NW = _NC * _NS
_RPW = _B // _NW
_NSTRIPE = 8
_HSTRIDE = 2048
_LEVELS = ((20, 2048), (10, 1024), (0, 1024))


def _body(x_hbm, out_hbm, x_v, hist_v, s_v):
    cid = lax.axis_index("c")
    sid = lax.axis_index("s")
    wid = sid * _NC + cid
    lanes = lax.broadcasted_iota(jnp.int32, (_L,), 0)
    stripe_base = (lanes & 7) * _HSTRIDE
    addend = jnp.where(lanes < 8, jnp.int32(1), jnp.int32(1 << 16))
    zero16 = jnp.zeros((_L,), jnp.int32)
    absmask = jnp.int32(0x7FFFFFFF)

    def u_of(j):
        x = x_v[pl.ds(j * _L, _L)]
        return x, lax.bitcast_convert_type(x, jnp.int32) & absmask

    for r in range(_RPW):
        row = wid * _RPW + r
        pltpu.sync_copy(x_hbm.at[row], x_v)

        pref = jnp.int32(0)
        rk = jnp.int32(_K)
        for li, (shift, nbuckets) in enumerate(_LEVELS):
            nch = nbuckets // _L
            bmask = jnp.int32(nbuckets - 1)
            maskshift = shift + (nbuckets - 1).bit_length()

            @plsc.parallel_loop(0, _NSTRIPE * nch, unroll=8)
            def _(t, nch=nch):
                l = t // nch
                j = t - l * nch
                hist_v[pl.ds(l * _HSTRIDE + j * _L, _L)] = zero16

            if li == 0:

                @plsc.parallel_loop(0, _NV, unroll=8)
                def _(j, shift=shift, bmask=bmask):
                    _, u = u_of(j)
                    bucket = (u >> shift) & bmask
                    plsc.addupdate_scatter(
                        hist_v, [stripe_base + bucket], addend)

            else:

                @plsc.parallel_loop(0, _NV, unroll=8)
                def _(j, shift=shift, bmask=bmask,
                      maskshift=maskshift, pref=pref):
                    _, u = u_of(j)
                    bucket = (u >> shift) & bmask
                    m = (u >> maskshift) == (pref >> maskshift)
                    plsc.addupdate_scatter(
                        hist_v, [stripe_base + bucket], addend, mask=m)

            @plsc.parallel_loop(
                0, nch, unroll=4, carry=(jnp.int32(0), zero16))
            def carry_out(jj, carry, nch=nch):
                c = nch - 1 - jj
                csum, cnt = carry
                acc = hist_v[pl.ds(0 * _HSTRIDE + c * _L, _L)]
                for l in range(1, _NSTRIPE):
                    acc = acc + hist_v[pl.ds(l * _HSTRIDE + c * _L, _L)]
                tot = (acc & jnp.int32(0xFFFF)) + (acc >> 16)
                s = plsc.cumsum(lax.rev(tot, dimensions=(0,)))
                s_desc = s + csum
                cnt = cnt + plsc.all_reduce_population_count(s_desc >= rk)
                s_v[pl.ds(c * _L, _L)] = lax.rev(s_desc, dimensions=(0,))
                return csum + jnp.sum(tot), cnt

            _, cntv = carry_out
            b = jnp.max(cntv) - 1
            s_v[pl.ds(nbuckets, _L)] = zero16
            above = jnp.max(plsc.load_gather(s_v, [zero16 + (b + 1)]))
            rk = rk - above
            pref = pref | (b << shift)

        thr = pref

        @plsc.parallel_loop(0, _NV, unroll=8)
        def _(j, thr=thr):
            x, u = u_of(j)
            x_v[pl.ds(j * _L, _L)] = jnp.where(u >= thr, x, jnp.float32(0.0))

        pltpu.sync_copy(x_v, out_hbm.at[row])


def kernel(X):
    mesh = plsc.VectorSubcoreMesh(
        core_axis_name="c", subcore_axis_name="s",
        num_cores=_NC, num_subcores=_NS)
    return pl.kernel(
        _body,
        out_type=jax.ShapeDtypeStruct((_B, _N), jnp.float32),
        mesh=mesh,
        scratch_types=[
            pltpu.VMEM((_N,), jnp.float32),
            pltpu.VMEM((_NSTRIPE * _HSTRIDE,), jnp.int32),
            pltpu.VMEM((2048 + _L,), jnp.int32),
        ],
        compiler_params=pltpu.CompilerParams(needs_layout_passes=False),
    )(X)


# speedup vs baseline: 4.0045x; 1.2042x over previous
import jax
import jax.numpy as jnp
from jax import lax
from jax.experimental import pallas as pl
from jax.experimental.pallas import tpu as pltpu
from jax.experimental.pallas import tpu_sc as plsc

_K = 2048
_N = 32768
_B = 128
_L = 16
_NV = _N // _L
_NC = 2
_NS = 16
_NW = _NC * _NS
_RPW = _B // _NW
_NSTRIPE = 8
_HSTRIDE = 2048
_LEVELS = ((20, 2048), (10, 1024), (0, 1024))


def _body(x_hbm, out_hbm, xa_v, xb_v, y_v, hist_v, s_v, sem_in, sem_out):
    cid = lax.axis_index("c")
    sid = lax.axis_index("s")
    wid = sid * _NC + cid
    lanes = lax.broadcasted_iota(jnp.int32, (_L,), 0)
    stripe_base = (lanes & 7) * _HSTRIDE
    addend = jnp.where(lanes < 8, jnp.int32(1), jnp.int32(1 << 16))
    zero16 = jnp.zeros((_L,), jnp.int32)
    absmask = jnp.int32(0x7FFFFFFF)
    bufs = (xa_v, xb_v)

    @plsc.parallel_loop(0, _NSTRIPE * (_HSTRIDE // _L), unroll=8)
    def _(t):
        hist_v[pl.ds(t * _L, _L)] = zero16

    in_cp = pltpu.async_copy(x_hbm.at[wid * _RPW], bufs[0], sem_in)
    out_cp = None

    for r in range(_RPW):
        row = wid * _RPW + r
        x_v = bufs[r % 2]
        in_cp.wait()
        if r + 1 < _RPW:
            in_cp = pltpu.async_copy(
                x_hbm.at[row + 1], bufs[(r + 1) % 2], sem_in)

        def u_of(j, x_v=x_v):
            x = x_v[pl.ds(j * _L, _L)]
            return x, lax.bitcast_convert_type(x, jnp.int32) & absmask

        pref = jnp.int32(0)
        rk = jnp.int32(_K)
        for li, (shift, nbuckets) in enumerate(_LEVELS):
            nch = nbuckets // _L
            bmask = jnp.int32(nbuckets - 1)
            maskshift = shift + (nbuckets - 1).bit_length()

            if li == 0:

                @plsc.parallel_loop(0, _NV, unroll=8)
                def _(j, shift=shift, bmask=bmask):
                    _, u = u_of(j)
                    bucket = (u >> shift) & bmask
                    plsc.addupdate_scatter(
                        hist_v, [stripe_base + bucket], addend)

            else:

                @plsc.parallel_loop(0, _NV, unroll=8)
                def _(j, shift=shift, bmask=bmask,
                      maskshift=maskshift, pref=pref):
                    _, u = u_of(j)
                    bucket = (u >> shift) & bmask
                    m = (u >> maskshift) == (pref >> maskshift)
                    plsc.addupdate_scatter(
                        hist_v, [stripe_base + bucket], addend, mask=m)

            @plsc.parallel_loop(
                0, nch, unroll=4, carry=(jnp.int32(0), zero16))
            def carry_out(jj, carry, nch=nch):
                c = nch - 1 - jj
                csum, cnt = carry
                acc = hist_v[pl.ds(0 * _HSTRIDE + c * _L, _L)]
                hist_v[pl.ds(0 * _HSTRIDE + c * _L, _L)] = zero16
                for l in range(1, _NSTRIPE):
                    acc = acc + hist_v[pl.ds(l * _HSTRIDE + c * _L, _L)]
                    hist_v[pl.ds(l * _HSTRIDE + c * _L, _L)] = zero16
                tot = (acc & jnp.int32(0xFFFF)) + (acc >> 16)
                s = plsc.cumsum(lax.rev(tot, dimensions=(0,)))
                s_desc = s + csum
                cnt = cnt + plsc.all_reduce_population_count(s_desc >= rk)
                s_v[pl.ds(c * _L, _L)] = lax.rev(s_desc, dimensions=(0,))
                return csum + jnp.sum(tot), cnt

            _, cntv = carry_out
            b = jnp.max(cntv) - 1
            s_v[pl.ds(nbuckets, _L)] = zero16
            above = jnp.max(plsc.load_gather(s_v, [zero16 + (b + 1)]))
            rk = rk - above
            pref = pref | (b << shift)

        thr = pref
        if out_cp is not None:
            out_cp.wait()

        @plsc.parallel_loop(0, _NV, unroll=8)
        def _(j, thr=thr):
            x, u = u_of(j)
            y_v[pl.ds(j * _L, _L)] = jnp.where(u >= thr, x, jnp.float32(0.0))

        out_cp = pltpu.async_copy(y_v, out_hbm.at[row], sem_out)

    out_cp.wait()


def kernel(X):
    mesh = plsc.VectorSubcoreMesh(
        core_axis_name="c", subcore_axis_name="s",
        num_cores=_NC, num_subcores=_NS)
    return pl.kernel(
        _body,
        out_type=jax.ShapeDtypeStruct((_B, _N), jnp.float32),
        mesh=mesh,
        scratch_types=[
            pltpu.VMEM((_N,), jnp.float32),
            pltpu.VMEM((_N,), jnp.float32),
            pltpu.VMEM((_N,), jnp.float32),
            pltpu.VMEM((_NSTRIPE * _HSTRIDE,), jnp.int32),
            pltpu.VMEM((2048 + _L,), jnp.int32),
            pltpu.SemaphoreType.DMA,
            pltpu.SemaphoreType.DMA,
        ],
        compiler_params=pltpu.CompilerParams(needs_layout_passes=False),
    )(X)
